# Initial kernel scaffold; baseline (speedup 1.0000x reference)
import jax
import jax.numpy as jnp
from jax.experimental import pallas as pl


def kernel(x, Wq, Wk, Wv):
    raise NotImplementedError("write your pallas kernel here")


# speedup vs baseline: 1.6479x; 1.6479x over previous
import math

import jax
import jax.numpy as jnp
from jax.experimental import pallas as pl

_B, _N, _C = 2, 4096, 1024
_SUB = 512
_SCALE = 1.0 / math.sqrt(_C)


def _attn_body(x_ref, wq_ref, wk_ref, wv_ref, os_ref, d_ref):
    xg = x_ref[0][:, :_C]
    q = jnp.dot(xg, wq_ref[...], preferred_element_type=jnp.float32)
    k = jnp.dot(xg, wk_ref[...], preferred_element_type=jnp.float32)
    v = jnp.dot(xg, wv_ref[...], preferred_element_type=jnp.float32)
    qb = q.astype(jnp.bfloat16)
    kb = k.astype(jnp.bfloat16)
    vb = v.astype(jnp.bfloat16)
    s = jax.lax.dot_general(
        qb, kb, (((1,), (1,)), ((), ())), preferred_element_type=jnp.float32
    ) * _SCALE
    e = jnp.exp(s)
    d = jnp.sum(e, axis=-1, keepdims=True)
    p = (e / d).astype(jnp.bfloat16)
    os = jax.lax.dot_general(
        p, vb, (((1,), (0,)), ((), ())), preferred_element_type=jnp.float32
    )
    os_ref[0] = os
    d_ref[0] = d


def _run_part(xv, wq, wk, wv, nseg, r, interpret=False):
    lanes = r * _C
    return pl.pallas_call(
        _attn_body,
        grid=(_B, nseg),
        in_specs=[
            pl.BlockSpec((1, _SUB, lanes), lambda b, s: (b, s, 0)),
            pl.BlockSpec((_C, _C), lambda b, s: (0, 0)),
            pl.BlockSpec((_C, _C), lambda b, s: (0, 0)),
            pl.BlockSpec((_C, _C), lambda b, s: (0, 0)),
        ],
        out_specs=[
            pl.BlockSpec((1, _SUB, _C), lambda b, s: (b, s, 0)),
            pl.BlockSpec((1, _SUB, 1), lambda b, s: (b, s, 0)),
        ],
        out_shape=[
            jax.ShapeDtypeStruct((_B, nseg * _SUB, _C), jnp.float32),
            jax.ShapeDtypeStruct((_B, nseg * _SUB, 1), jnp.float32),
        ],
        interpret=interpret,
    )(xv, wq, wk, wv)


def _mix_body(osa_ref, da_ref, osb_ref, db_ref, osc_ref, dc_ref, out_ref):
    a = da_ref[0]
    i2 = jax.lax.broadcasted_iota(jnp.int32, (_SUB, _SUB // 2), 0)
    j2 = jax.lax.broadcasted_iota(jnp.int32, (_SUB, _SUB // 2), 1)
    s2 = (i2 == 2 * j2).astype(jnp.float32)
    i8 = jax.lax.broadcasted_iota(jnp.int32, (_SUB, _SUB // 8), 0)
    j8 = jax.lax.broadcasted_iota(jnp.int32, (_SUB, _SUB // 8), 1)
    s8 = (i8 == 8 * j8).astype(jnp.float32)
    db = db_ref[0]
    dc = dc_ref[0]
    mb = jnp.concatenate(
        [db * osb_ref[0], jnp.broadcast_to(db, (_SUB // 2, 128))], axis=1
    )
    mc = jnp.concatenate(
        [dc * osc_ref[0], jnp.broadcast_to(dc, (_SUB // 8, 128))], axis=1
    )
    sb = jnp.dot(s2, mb, preferred_element_type=jnp.float32,
                 precision=jax.lax.Precision.HIGHEST)
    sc = jnp.dot(s8, mc, preferred_element_type=jnp.float32,
                 precision=jax.lax.Precision.HIGHEST)
    num = a * osa_ref[0] + sb[:, :_C] + sc[:, :_C]
    ds = a + sb[:, _C:_C + 1] + sc[:, _C:_C + 1]
    out_ref[0] = num / ds


def _mix(osa, da, osb, db, osc, dc, interpret=False):
    return pl.pallas_call(
        _mix_body,
        grid=(_B, _N // _SUB),
        in_specs=[
            pl.BlockSpec((1, _SUB, _C), lambda b, k: (b, k, 0)),
            pl.BlockSpec((1, _SUB, 1), lambda b, k: (b, k, 0)),
            pl.BlockSpec((1, _SUB // 2, _C), lambda b, k: (b, k, 0)),
            pl.BlockSpec((1, _SUB // 2, 1), lambda b, k: (b, k, 0)),
            pl.BlockSpec((1, _SUB // 8, _C), lambda b, k: (b, k, 0)),
            pl.BlockSpec((1, _SUB // 8, 1), lambda b, k: (b, k, 0)),
        ],
        out_specs=pl.BlockSpec((1, _SUB, _C), lambda b, k: (b, k, 0)),
        out_shape=jax.ShapeDtypeStruct((_B, _N, _C), jnp.float32),
        interpret=interpret,
    )(osa, da, osb, db, osc, dc)


def _dilated_attention(x, wq, wk, wv, interpret=False):
    xb = x.astype(jnp.bfloat16)
    wqb = wq.astype(jnp.bfloat16)
    wkb = wk.astype(jnp.bfloat16)
    wvb = wv.astype(jnp.bfloat16)
    osa, da = _run_part(xb, wqb, wkb, wvb, 8, 1, interpret)
    osb, db = _run_part(
        xb.reshape(_B, _N // 2, 2 * _C), wqb, wkb, wvb, 4, 2, interpret
    )
    osc, dc = _run_part(
        xb.reshape(_B, _N // 8, 8 * _C), wqb, wkb, wvb, 1, 8, interpret
    )
    return _mix(osa, da, osb, db, osc, dc, interpret)


def kernel(x, Wq, Wk, Wv):
    return _dilated_attention(x, Wq, Wk, Wv)
